# Initial kernel scaffold; baseline (speedup 1.0000x reference)
import jax
import jax.numpy as jnp
from jax.experimental import pallas as pl


def kernel(sparse_idx, dense, tables, w_dense, bias):
    raise NotImplementedError("write your pallas kernel here")


# speedup vs baseline: 1.1477x; 1.1477x over previous
import functools

import jax
import jax.numpy as jnp
from jax import lax
from jax.experimental import pallas as pl
from jax.experimental.pallas import tpu as pltpu
from jax.experimental.pallas import tpu_sc as plsc

_LANES = 16
_CHUNK = 8192


def _make_sc_gather(F, B, V):
    mesh = plsc.VectorSubcoreMesh(core_axis_name="c", subcore_axis_name="s")
    num_subcores = 16

    @functools.partial(
        pl.kernel,
        out_type=jax.ShapeDtypeStruct((F, B), jnp.float32),
        mesh=mesh,
        compiler_params=pltpu.CompilerParams(needs_layout_passes=False),
        scratch_types=[
            pltpu.VMEM((V,), jnp.float32),
            pltpu.VMEM((_CHUNK,), jnp.int32),
            pltpu.VMEM((_CHUNK,), jnp.float32),
        ],
    )
    def sc_gather(tables_hbm, idx_hbm, out_hbm, tbl_v, idx_v, g_v):
        c = lax.axis_index("c")
        s = lax.axis_index("s")
        wid = c * num_subcores + s

        @pl.when(wid < F)
        def _():
            pltpu.sync_copy(tables_hbm.at[wid], tbl_v)
            for ci in range(B // _CHUNK):
                pltpu.sync_copy(idx_hbm.at[wid, pl.ds(ci * _CHUNK, _CHUNK)], idx_v)

                def body(j, carry):
                    iv = idx_v[pl.ds(j * _LANES, _LANES)]
                    g_v[pl.ds(j * _LANES, _LANES)] = plsc.load_gather(tbl_v, [iv])
                    return carry

                lax.fori_loop(0, _CHUNK // _LANES, body, 0)
                pltpu.sync_copy(g_v, out_hbm.at[wid, pl.ds(ci * _CHUNK, _CHUNK)])

    return sc_gather


def _tc_finish(partials, dense, w2d, bias2d):
    B = partials.shape[1]

    def body(p_ref, d_ref, w_ref, b_ref, o_ref):
        s = jnp.sum(p_ref[...], axis=0, keepdims=True)
        dm = lax.dot_general(
            w_ref[...], d_ref[...],
            dimension_numbers=(((1,), (1,)), ((), ())),
            preferred_element_type=jnp.float32,
        )
        o_ref[...] = jax.nn.sigmoid(s + dm + b_ref[...])

    return pl.pallas_call(
        body,
        out_shape=jax.ShapeDtypeStruct((1, B), jnp.float32),
    )(partials, dense, w2d, bias2d)


def kernel(sparse_idx, dense, tables, w_dense, bias):
    B, F = sparse_idx.shape
    V = tables.shape[1]
    idx_t = sparse_idx.T.astype(jnp.int32)
    partials = _make_sc_gather(F, B, V)(tables, idx_t)
    out2d = _tc_finish(partials, dense, w_dense.reshape(1, -1), bias.reshape(1, 1))
    return out2d.reshape(B)


# speedup vs baseline: 1.2546x; 1.0932x over previous
import functools

import jax
import jax.numpy as jnp
from jax import lax
from jax.experimental import pallas as pl
from jax.experimental.pallas import tpu as pltpu
from jax.experimental.pallas import tpu_sc as plsc

_LANES = 16
_CHUNK = 8192
_UNROLL = 8
_TCBLK = 2048


def _make_sc_gather(F, B, V):
    mesh = plsc.VectorSubcoreMesh(core_axis_name="c", subcore_axis_name="s")
    num_subcores = 16

    @functools.partial(
        pl.kernel,
        out_type=jax.ShapeDtypeStruct((F, B), jnp.float32),
        mesh=mesh,
        compiler_params=pltpu.CompilerParams(needs_layout_passes=False),
        scratch_types=[
            pltpu.VMEM((V,), jnp.float32),
            pltpu.VMEM((_CHUNK,), jnp.int32),
            pltpu.VMEM((_CHUNK,), jnp.float32),
        ],
    )
    def sc_gather(tables_hbm, idx_hbm, out_hbm, tbl_v, idx_v, g_v):
        c = lax.axis_index("c")
        s = lax.axis_index("s")
        wid = c * num_subcores + s

        @pl.when(wid < F)
        def _():
            pltpu.sync_copy(tables_hbm.at[wid], tbl_v)
            for ci in range(B // _CHUNK):
                pltpu.sync_copy(idx_hbm.at[wid, pl.ds(ci * _CHUNK, _CHUNK)],
                                idx_v)

                def body(j, carry):
                    base = j * (_LANES * _UNROLL)
                    for u in range(_UNROLL):
                        iv = idx_v[pl.ds(base + u * _LANES, _LANES)]
                        g_v[pl.ds(base + u * _LANES, _LANES)] = (
                            plsc.load_gather(tbl_v, [iv]))
                    return carry

                lax.fori_loop(0, _CHUNK // (_LANES * _UNROLL), body, 0)
                pltpu.sync_copy(g_v, out_hbm.at[wid, pl.ds(ci * _CHUNK,
                                                           _CHUNK)])

    return sc_gather


def _tc_finish(partials, dense_t, w2d, bias2d):
    F, B = partials.shape
    D = dense_t.shape[0]

    def body(p_ref, d_ref, w_ref, b_ref, o_ref):
        sc_sum = jnp.sum(p_ref[...], axis=0, keepdims=True)
        dm = jnp.sum(d_ref[...] * w_ref[...], axis=0, keepdims=True)
        o_ref[...] = jax.nn.sigmoid(sc_sum + dm + b_ref[...])

    grid = (B // _TCBLK,)
    return pl.pallas_call(
        body,
        grid=grid,
        in_specs=[
            pl.BlockSpec((F, _TCBLK), lambda j: (0, j)),
            pl.BlockSpec((D, _TCBLK), lambda j: (0, j)),
            pl.BlockSpec((D, 1), lambda j: (0, 0)),
            pl.BlockSpec((1, 1), lambda j: (0, 0)),
        ],
        out_specs=pl.BlockSpec((1, _TCBLK), lambda j: (0, j)),
        out_shape=jax.ShapeDtypeStruct((1, B), jnp.float32),
    )(partials, dense_t, w2d, bias2d)


def kernel(sparse_idx, dense, tables, w_dense, bias):
    B, F = sparse_idx.shape
    V = tables.shape[1]
    idx_t = sparse_idx.T.astype(jnp.int32)
    dense_t = dense.T
    partials = _make_sc_gather(F, B, V)(tables, idx_t)
    out2d = _tc_finish(partials, dense_t, w_dense.reshape(-1, 1),
                       bias.reshape(1, 1))
    return out2d.reshape(B)
